# baseline (speedup 1.0000x reference)
import functools

import jax
import jax.numpy as jnp
from jax import lax
from jax.experimental import pallas as pl
from jax.experimental.pallas import tpu as pltpu
from jax.experimental.pallas import tpu_sc as plsc

VAR_IDX = 3
D = 32
NC = 2
NS = 16
NW = NC * NS
SUB = 128
SPC = 4
CHUNK = SUB * SPC
L = 16


def _make_gather(S, Bt):
    B = S * Bt
    b_per_w = B // NW
    n_idx_rows = b_per_w // SUB
    cpw = b_per_w // CHUNK
    n_pairs = cpw // 2
    q_per_s = Bt // CHUNK
    mesh = plsc.VectorSubcoreMesh(core_axis_name="c", subcore_axis_name="s")

    @functools.partial(
        pl.kernel,
        mesh=mesh,
        out_type=jax.ShapeDtypeStruct((S, D, Bt), jnp.float32),
        scratch_types=[
            pltpu.VMEM((n_idx_rows, SUB), jnp.int32),
            pltpu.VMEM((CHUNK, D), jnp.float32),
            pltpu.VMEM((CHUNK, D), jnp.float32),
            pltpu.VMEM((D, CHUNK), jnp.float32),
            pltpu.VMEM((D, CHUNK), jnp.float32),
            pltpu.SemaphoreType.DMA,
            pltpu.SemaphoreType.DMA,
            pltpu.SemaphoreType.DMA,
            pltpu.SemaphoreType.DMA,
        ],
        compiler_params=pltpu.CompilerParams(
            use_tc_tiling_on_sc=False, needs_layout_passes=False),
    )
    def body(idx_hbm, table_hbm, out_hbm, idx_v, g0, g1, t0, t1,
             gsem0, gsem1, osem0, osem1):
        wid = lax.axis_index("s") * NC + lax.axis_index("c")
        h0 = wid * cpw
        pltpu.sync_copy(idx_hbm.at[wid], idx_v)
        iota16 = lax.iota(jnp.int32, L)

        def fire_g(hl, g, gsem):
            for k in range(SPC):
                pltpu.async_copy(
                    table_hbm.at[idx_v.at[hl * SPC + k]],
                    g.at[pl.ds(k * SUB, SUB)],
                    gsem,
                )

        def wait_g(g, gsem):
            pltpu.make_async_copy(
                table_hbm.at[pl.ds(0, CHUNK)], g, gsem).wait()

        def transpose(g, t):
            def col_block(gi, carry):
                rows = gi * L + iota16
                for d in range(D):
                    cols = jnp.full((L,), d, jnp.int32)
                    t[d, pl.ds(gi * L, L)] = plsc.load_gather(g, [rows, cols])
                return carry
            lax.fori_loop(0, CHUNK // L, col_block, 0)

        def fire_w(hl, t, osem):
            h = h0 + hl
            s = h // q_per_s
            b0 = (h % q_per_s) * CHUNK
            pltpu.async_copy(t, out_hbm.at[s, :, pl.ds(b0, CHUNK)], osem)

        def wait_w(t, osem):
            pltpu.make_async_copy(
                t, out_hbm.at[0, :, pl.ds(0, CHUNK)], osem).wait()

        fire_g(0, g0, gsem0)

        def pair(j, carry):
            hl = 2 * j
            wait_g(g0, gsem0)
            fire_g(hl + 1, g1, gsem1)

            @pl.when(j > 0)
            def _():
                wait_w(t0, osem0)
            transpose(g0, t0)
            fire_w(hl, t0, osem0)

            wait_g(g1, gsem1)

            @pl.when(j < n_pairs - 1)
            def _():
                fire_g(hl + 2, g0, gsem0)

            @pl.when(j > 0)
            def _():
                wait_w(t1, osem1)
            transpose(g1, t1)
            fire_w(hl + 1, t1, osem1)
            return carry

        lax.fori_loop(0, n_pairs, pair, 0)
        wait_w(t0, osem0)
        wait_w(t1, osem1)

    return body


def kernel(X, table):
    Bt, S, _ = X.shape
    idx3 = jnp.transpose(X, (2, 1, 0))[VAR_IDX].reshape(NW, S * Bt // (NW * SUB), SUB)
    out3 = _make_gather(S, Bt)(idx3, table)
    return jnp.transpose(out3, (2, 0, 1))


# speedup vs baseline: 1.0006x; 1.0006x over previous
import functools

import jax
import jax.numpy as jnp
from jax import lax
from jax.experimental import pallas as pl
from jax.experimental.pallas import tpu as pltpu
from jax.experimental.pallas import tpu_sc as plsc

VAR_IDX = 3
D = 32
NC = 2
NS = 16
NW = NC * NS
SUB = 128
SPC = 4
CHUNK = SUB * SPC
L = 16


def _make_gather(S, Bt):
    B = S * Bt
    b_per_w = B // NW
    n_idx_rows = b_per_w // SUB
    cpw = b_per_w // CHUNK
    n_pairs = cpw // 2
    q_per_s = Bt // CHUNK
    mesh = plsc.VectorSubcoreMesh(core_axis_name="c", subcore_axis_name="s")

    @functools.partial(
        pl.kernel,
        mesh=mesh,
        out_type=jax.ShapeDtypeStruct((S, D, Bt), jnp.float32),
        scratch_types=[
            pltpu.VMEM((n_idx_rows, SUB), jnp.int32),
            pltpu.VMEM((CHUNK, D), jnp.float32),
            pltpu.VMEM((CHUNK, D), jnp.float32),
            pltpu.VMEM((D, CHUNK), jnp.float32),
            pltpu.VMEM((D, CHUNK), jnp.float32),
            pltpu.SemaphoreType.DMA,
            pltpu.SemaphoreType.DMA,
            pltpu.SemaphoreType.DMA,
            pltpu.SemaphoreType.DMA,
        ],
        compiler_params=pltpu.CompilerParams(
            use_tc_tiling_on_sc=False, needs_layout_passes=False,
            disable_bounds_checks=True),
    )
    def body(idx_hbm, table_hbm, out_hbm, idx_v, g0, g1, t0, t1,
             gsem0, gsem1, osem0, osem1):
        wid = lax.axis_index("s") * NC + lax.axis_index("c")
        h0 = wid * cpw
        pltpu.sync_copy(idx_hbm.at[wid], idx_v)
        iota16 = lax.iota(jnp.int32, L)

        def fire_g(hl, g, gsem):
            for k in range(SPC):
                pltpu.async_copy(
                    table_hbm.at[idx_v.at[hl * SPC + k]],
                    g.at[pl.ds(k * SUB, SUB)],
                    gsem,
                )

        def wait_g(g, gsem):
            pltpu.make_async_copy(
                table_hbm.at[pl.ds(0, CHUNK)], g, gsem).wait()

        cols_d = [jnp.full((L,), d, jnp.int32) for d in range(D)]

        def transpose(g, t):
            def col_block(gi, carry):
                rows = gi * L + iota16
                for d in range(D):
                    t[d, pl.ds(gi * L, L)] = plsc.load_gather(g, [rows, cols_d[d]])
                return carry
            lax.fori_loop(0, CHUNK // L, col_block, 0)

        def fire_w(hl, t, osem):
            h = h0 + hl
            s = h // q_per_s
            b0 = (h % q_per_s) * CHUNK
            pltpu.async_copy(t, out_hbm.at[s, :, pl.ds(b0, CHUNK)], osem)

        def wait_w(t, osem):
            pltpu.make_async_copy(
                t, out_hbm.at[0, :, pl.ds(0, CHUNK)], osem).wait()

        fire_g(0, g0, gsem0)

        def pair(j, carry):
            hl = 2 * j
            wait_g(g0, gsem0)
            fire_g(hl + 1, g1, gsem1)

            @pl.when(j > 0)
            def _():
                wait_w(t0, osem0)
            transpose(g0, t0)
            fire_w(hl, t0, osem0)

            wait_g(g1, gsem1)

            @pl.when(j < n_pairs - 1)
            def _():
                fire_g(hl + 2, g0, gsem0)

            @pl.when(j > 0)
            def _():
                wait_w(t1, osem1)
            transpose(g1, t1)
            fire_w(hl + 1, t1, osem1)
            return carry

        lax.fori_loop(0, n_pairs, pair, 0)
        wait_w(t0, osem0)
        wait_w(t1, osem1)

    return body


def kernel(X, table):
    Bt, S, _ = X.shape
    idx3 = jnp.transpose(X, (2, 1, 0))[VAR_IDX].reshape(NW, S * Bt // (NW * SUB), SUB)
    out3 = _make_gather(S, Bt)(idx3, table)
    return jnp.transpose(out3, (2, 0, 1))
